# Initial kernel scaffold; baseline (speedup 1.0000x reference)
import jax
import jax.numpy as jnp
from jax.experimental import pallas as pl


def kernel(x, edge_index, edge_attr, batch, lin_W, lin_b, edge_W, edge_b, root_W, conv_b, gru_Wih, gru_Whh, gru_bih, gru_bhh, lstm_Wih, lstm_Whh, lstm_bih, lstm_bhh, sp_W, sp_b, prelu_a):
    raise NotImplementedError("write your pallas kernel here")


# speedup vs baseline: 3.2661x; 3.2661x over previous
import functools

import jax
import jax.numpy as jnp
from jax import lax
from jax.experimental import pallas as pl
from jax.experimental.pallas import tpu as pltpu
from jax.experimental.pallas import tpu_sc as plsc

NUM_GRAPHS = 256
GW = 128
NW = 32


def _node_body(x_ref, w_ref, b_ref, o_ref):
    o_ref[...] = jnp.maximum(x_ref[...] @ w_ref[...] + b_ref[...], 0.0)


def _msg_body(ea_ref, fs_ref, w2_ref, o_ref):
    ea = ea_ref[...]
    fs = fs_ref[...]
    k1 = ea.shape[1]
    h = fs.shape[1]
    ii = lax.broadcasted_iota(jnp.int32, (k1, k1 * h), 0)
    jj = lax.broadcasted_iota(jnp.int32, (k1, k1 * h), 1)
    rm = (jj // h == ii).astype(jnp.float32)
    p = (ea @ rm) * jnp.concatenate([fs] * k1, axis=1)
    o_ref[...] = p @ w2_ref[...]


def _gru_body(aggp_ref, f_ref, h_ref, rw_ref, cb_ref, wih_ref, whh_ref,
              bih_ref, bhh_ref, o_ref):
    n = f_ref.shape[0]
    hdim = f_ref.shape[1]
    agg = aggp_ref[0, :n, :] + aggp_ref[1, :n, :]
    f = f_ref[...]
    h = h_ref[...]
    conv = agg + f @ rw_ref[...] + cb_ref[...]
    a = jnp.maximum(conv, 0.0)
    gi = a @ wih_ref[...] + bih_ref[...]
    gh = h @ whh_ref[...] + bhh_ref[...]
    r = jax.nn.sigmoid(gi[:, :hdim] + gh[:, :hdim])
    z = jax.nn.sigmoid(gi[:, hdim:2 * hdim] + gh[:, hdim:2 * hdim])
    nn = jnp.tanh(gi[:, 2 * hdim:] + r * gh[:, 2 * hdim:])
    o_ref[...] = (1.0 - z) * nn + z * h


def _s2s_body(na_ref, naT_ref, b_ref, wih_ref, whh_ref, lb_ref,
              spw_ref, spb_ref, pa_ref, o_ref, num_steps):
    na = na_ref[...]
    naT = naT_ref[...]
    n = na.shape[0]
    ic = na.shape[1]
    g = NUM_GRAPHS
    brow = b_ref[...]
    gid = lax.broadcasted_iota(jnp.int32, (g, 1), 0)
    ot = brow == gid
    q_star = jnp.zeros((g, 2 * ic), jnp.float32)
    h_l = jnp.zeros((g, ic), jnp.float32)
    c_l = jnp.zeros((g, ic), jnp.float32)
    for _ in range(num_steps):
        gates = q_star @ wih_ref[...] + h_l @ whh_ref[...] + lb_ref[...]
        g_i = gates[:, :ic]
        g_f = gates[:, ic:2 * ic]
        g_g = gates[:, 2 * ic:3 * ic]
        g_o = gates[:, 3 * ic:]
        c_l = jax.nn.sigmoid(g_f) * c_l + jax.nn.sigmoid(g_i) * jnp.tanh(g_g)
        h_l = jax.nn.sigmoid(g_o) * jnp.tanh(c_l)
        d = h_l @ naT
        em = jnp.max(jnp.where(ot, d, -1e30), axis=1, keepdims=True)
        em = jnp.where(em > -1e29, em, 0.0)
        e2 = jnp.exp(jnp.where(ot, d - em, -1e30))
        den = jnp.sum(e2, axis=1, keepdims=True)
        alpha = e2 / (den + 1e-16)
        r_vec = alpha @ na
        q_star = jnp.concatenate([h_l, r_vec], axis=1)
    out = q_star @ spw_ref[...] + spb_ref[...]
    o_ref[...] = jnp.where(out > 0, out, pa_ref[...] * out)


def _sc_gather(table, idx2):
    nt, h = table.shape
    r = idx2.shape[0]
    rpw = r // NW
    mesh = plsc.VectorSubcoreMesh(core_axis_name="core",
                                  subcore_axis_name="subcore")

    @functools.partial(
        pl.kernel,
        out_type=jax.ShapeDtypeStruct((r, GW, h), jnp.float32),
        mesh=mesh,
        scratch_types=[pltpu.VMEM((rpw, GW), jnp.int32),
                       pltpu.VMEM((rpw, GW, h), jnp.float32)],
        compiler_params=pltpu.CompilerParams(use_tc_tiling_on_sc=False),
    )
    def k(tab_hbm, idx_hbm, o_hbm, idx_v, rows_v):
        cid = lax.axis_index("core")
        sid = lax.axis_index("subcore")
        wid = sid * 2 + cid
        r0 = wid * rpw
        pltpu.sync_copy(idx_hbm.at[pl.ds(r0, rpw)], idx_v)

        @pl.loop(0, rpw)
        def _(j):
            pltpu.sync_copy(tab_hbm.at[idx_v.at[j]], rows_v.at[j])

        pltpu.sync_copy(rows_v, o_hbm.at[pl.ds(r0, rpw)])

    return k(table, idx2)


def _sc_scatter_add(msg3, idx2, np_rows):
    r, _, h = msg3.shape
    rpw = r // NW
    zr = np_rows // 16
    mesh = plsc.VectorSubcoreMesh(core_axis_name="core",
                                  subcore_axis_name="subcore")

    @functools.partial(
        pl.kernel,
        out_type=jax.ShapeDtypeStruct((2, np_rows, h), jnp.float32),
        mesh=mesh,
        scratch_types=[pltpu.VMEM((rpw, GW), jnp.int32),
                       pltpu.VMEM((rpw, GW, h), jnp.float32),
                       pltpu.VMEM((zr, h), jnp.float32),
                       pltpu.VMEM_SHARED((np_rows, h), jnp.float32)],
        compiler_params=pltpu.CompilerParams(use_tc_tiling_on_sc=False),
    )
    def k(msg_hbm, idx_hbm, o_hbm, idx_v, msg_v, zb, acc):
        cid = lax.axis_index("core")
        sid = lax.axis_index("subcore")
        wid = sid * 2 + cid

        @pl.loop(0, zr)
        def _(i):
            zb[i] = jnp.zeros((h,), jnp.float32)

        pltpu.sync_copy(zb, acc.at[pl.ds(sid * zr, zr)])
        plsc.subcore_barrier()
        r0 = wid * rpw
        pltpu.sync_copy(idx_hbm.at[pl.ds(r0, rpw)], idx_v)
        pltpu.sync_copy(msg_hbm.at[pl.ds(r0, rpw)], msg_v)

        @pl.loop(0, rpw)
        def _(j):
            pltpu.sync_copy(msg_v.at[j], acc.at[idx_v.at[j]], add=True)
        plsc.subcore_barrier()
        pltpu.sync_copy(acc.at[pl.ds(sid * zr, zr)],
                        o_hbm.at[cid, pl.ds(sid * zr, zr)])

    return k(msg3, idx2)


def kernel(x, edge_index, edge_attr, batch, lin_W, lin_b, edge_W, edge_b,
           root_W, conv_b, gru_Wih, gru_Whh, gru_bih, gru_bhh,
           lstm_Wih, lstm_Whh, lstm_bih, lstm_bhh, sp_W, sp_b, prelu_a):
    f32 = jnp.float32
    n, node_in = x.shape
    e = edge_index.shape[1]
    h = lin_W.shape[1]
    k_in = edge_attr.shape[1]
    g = NUM_GRAPHS
    out_dim = sp_W.shape[1]
    ic = 2 * h

    chunk = NW * GW
    epad = ((e + chunk - 1) // chunk) * chunk
    pad = epad - e
    np_rows = ((n + 1 + 15) // 16) * 16
    src = edge_index[0]
    dst = edge_index[1]
    src2 = jnp.concatenate([src, jnp.zeros((pad,), jnp.int32)]).reshape(-1, GW)
    dst2 = jnp.concatenate([dst, jnp.full((pad,), n, jnp.int32)]).reshape(-1, GW)
    ea1 = jnp.concatenate([edge_attr, jnp.ones((e, 1), f32)], axis=1)
    ea_p = jnp.concatenate([ea1, jnp.zeros((pad, k_in + 1), f32)], axis=0)
    w2 = jnp.concatenate([edge_W, edge_b.reshape(1, h * h)],
                         axis=0).reshape((k_in + 1) * h, h)
    wihT = gru_Wih.T
    whhT = gru_Whh.T
    bih2 = gru_bih.reshape(1, 3 * h)
    bhh2 = gru_bhh.reshape(1, 3 * h)
    lwihT = lstm_Wih.T
    lwhhT = lstm_Whh.T
    lb = (lstm_bih + lstm_bhh).reshape(1, 4 * ic)
    pa2 = jnp.broadcast_to(prelu_a.reshape(1, 1), (1, out_dim))

    node = pl.pallas_call(
        _node_body,
        out_shape=jax.ShapeDtypeStruct((n, h), f32),
    )(x, lin_W, lin_b.reshape(1, h))

    blk = 4096
    msg_call = pl.pallas_call(
        _msg_body,
        grid=(epad // blk,),
        in_specs=[
            pl.BlockSpec((blk, k_in + 1), lambda i: (i, 0)),
            pl.BlockSpec((blk, h), lambda i: (i, 0)),
            pl.BlockSpec(((k_in + 1) * h, h), lambda i: (0, 0)),
        ],
        out_specs=pl.BlockSpec((blk, h), lambda i: (i, 0)),
        out_shape=jax.ShapeDtypeStruct((epad, h), f32),
    )
    gru_call = pl.pallas_call(
        _gru_body,
        out_shape=jax.ShapeDtypeStruct((n, h), f32),
    )

    feats = node
    for _ in range(3):
        fs3 = _sc_gather(feats, src2)
        msg = msg_call(ea_p, fs3.reshape(epad, h), w2)
        aggp = _sc_scatter_add(msg.reshape(-1, GW, h), dst2, np_rows)
        feats = gru_call(aggp, feats, feats, root_W, conv_b.reshape(1, h),
                         wihT, whhT, bih2, bhh2)

    na = jnp.concatenate([node, feats], axis=1)
    out = pl.pallas_call(
        functools.partial(_s2s_body, num_steps=3),
        out_shape=jax.ShapeDtypeStruct((g, out_dim), f32),
    )(na, na.T, batch.reshape(1, n), lwihT, lwhhT, lb,
      sp_W, sp_b.reshape(1, out_dim), pa2)
    return out


# speedup vs baseline: 3.3410x; 1.0229x over previous
import functools

import jax
import jax.numpy as jnp
from jax import lax
from jax.experimental import pallas as pl
from jax.experimental.pallas import tpu as pltpu
from jax.experimental.pallas import tpu_sc as plsc

NUM_GRAPHS = 256
GW = 128
NW = 32


def _node_body(x_ref, w_ref, b_ref, o_ref):
    o_ref[...] = jnp.maximum(x_ref[...] @ w_ref[...] + b_ref[...], 0.0)


def _msg_body(ea_ref, fs_ref, rm_ref, w2_ref, o_ref):
    ea = ea_ref[...]
    fs = fs_ref[...]
    k1 = ea.shape[1]
    p = jnp.dot(ea, rm_ref[...],
                preferred_element_type=jnp.float32)
    p = p * jnp.concatenate([fs] * k1, axis=1)
    o_ref[...] = jnp.dot(p, w2_ref[...],
                         preferred_element_type=jnp.float32)


def _gru_body(aggp_ref, f_ref, h_ref, rw_ref, cb_ref, wih_ref, whh_ref,
              bih_ref, bhh_ref, o_ref):
    n = f_ref.shape[0]
    hdim = f_ref.shape[1]
    agg = aggp_ref[0, :n, :] + aggp_ref[1, :n, :]
    f = f_ref[...]
    h = h_ref[...]
    conv = agg + f @ rw_ref[...] + cb_ref[...]
    a = jnp.maximum(conv, 0.0)
    gi = a @ wih_ref[...] + bih_ref[...]
    gh = h @ whh_ref[...] + bhh_ref[...]
    r = jax.nn.sigmoid(gi[:, :hdim] + gh[:, :hdim])
    z = jax.nn.sigmoid(gi[:, hdim:2 * hdim] + gh[:, hdim:2 * hdim])
    nn = jnp.tanh(gi[:, 2 * hdim:] + r * gh[:, 2 * hdim:])
    o_ref[...] = (1.0 - z) * nn + z * h


def _s2s_body(na_ref, naT_ref, b_ref, wih_ref, whh_ref, lb_ref,
              spw_ref, spb_ref, pa_ref, o_ref, num_steps):
    na = na_ref[...]
    naT = naT_ref[...]
    n = na.shape[0]
    ic = na.shape[1]
    g = NUM_GRAPHS
    brow = b_ref[...]
    gid = lax.broadcasted_iota(jnp.int32, (g, 1), 0)
    ot = brow == gid
    q_star = jnp.zeros((g, 2 * ic), jnp.float32)
    h_l = jnp.zeros((g, ic), jnp.float32)
    c_l = jnp.zeros((g, ic), jnp.float32)
    for _ in range(num_steps):
        gates = q_star @ wih_ref[...] + h_l @ whh_ref[...] + lb_ref[...]
        g_i = gates[:, :ic]
        g_f = gates[:, ic:2 * ic]
        g_g = gates[:, 2 * ic:3 * ic]
        g_o = gates[:, 3 * ic:]
        c_l = jax.nn.sigmoid(g_f) * c_l + jax.nn.sigmoid(g_i) * jnp.tanh(g_g)
        h_l = jax.nn.sigmoid(g_o) * jnp.tanh(c_l)
        d = h_l @ naT
        em = jnp.max(jnp.where(ot, d, -1e30), axis=1, keepdims=True)
        em = jnp.where(em > -1e29, em, 0.0)
        e2 = jnp.exp(jnp.where(ot, d - em, -1e30))
        den = jnp.sum(e2, axis=1, keepdims=True)
        alpha = e2 / (den + 1e-16)
        r_vec = alpha @ na
        q_star = jnp.concatenate([h_l, r_vec], axis=1)
    out = q_star @ spw_ref[...] + spb_ref[...]
    o_ref[...] = jnp.where(out > 0, out, pa_ref[...] * out)


_GRP = 8


def _sc_gather(table, idx2):
    nt, h = table.shape
    r = idx2.shape[0]
    rpw = r // NW
    mesh = plsc.VectorSubcoreMesh(core_axis_name="core",
                                  subcore_axis_name="subcore")

    @functools.partial(
        pl.kernel,
        out_type=jax.ShapeDtypeStruct((r * GW, h), jnp.float32),
        mesh=mesh,
        scratch_types=[pltpu.VMEM((rpw, GW), jnp.int32),
                       pltpu.VMEM((rpw * GW, h), jnp.float32),
                       pltpu.SemaphoreType.DMA],
        compiler_params=pltpu.CompilerParams(use_tc_tiling_on_sc=False),
    )
    def k(tab_hbm, idx_hbm, o_hbm, idx_v, rows_v, sem):
        cid = lax.axis_index("core")
        sid = lax.axis_index("subcore")
        wid = sid * 2 + cid
        r0 = wid * rpw
        pltpu.sync_copy(idx_hbm.at[pl.ds(r0, rpw)], idx_v)

        @pl.loop(0, rpw // _GRP)
        def _(t):
            cps = [pltpu.async_copy(
                tab_hbm.at[idx_v.at[t * _GRP + u]],
                rows_v.at[pl.ds((t * _GRP + u) * GW, GW)], sem)
                for u in range(_GRP)]
            for c in cps:
                c.wait()

        pltpu.sync_copy(rows_v, o_hbm.at[pl.ds(r0 * GW, rpw * GW)])

    return k(table, idx2)


def _sc_scatter_add(msg3, idx2, np_rows):
    r, _, h = msg3.shape
    rpw = r // NW
    zr = np_rows // 16
    mesh = plsc.VectorSubcoreMesh(core_axis_name="core",
                                  subcore_axis_name="subcore")

    @functools.partial(
        pl.kernel,
        out_type=jax.ShapeDtypeStruct((2, np_rows, h), jnp.float32),
        mesh=mesh,
        scratch_types=[pltpu.VMEM((rpw, GW), jnp.int32),
                       pltpu.VMEM((rpw, GW, h), jnp.float32),
                       pltpu.VMEM((zr, h), jnp.float32),
                       pltpu.VMEM_SHARED((np_rows, h), jnp.float32)],
        compiler_params=pltpu.CompilerParams(use_tc_tiling_on_sc=False),
    )
    def k(msg_hbm, idx_hbm, o_hbm, idx_v, msg_v, zb, acc):
        cid = lax.axis_index("core")
        sid = lax.axis_index("subcore")
        wid = sid * 2 + cid

        @pl.loop(0, zr)
        def _(i):
            zb[i] = jnp.zeros((h,), jnp.float32)

        pltpu.sync_copy(zb, acc.at[pl.ds(sid * zr, zr)])
        plsc.subcore_barrier()
        r0 = wid * rpw
        pltpu.sync_copy(idx_hbm.at[pl.ds(r0, rpw)], idx_v)
        pltpu.sync_copy(msg_hbm.at[pl.ds(r0, rpw)], msg_v)

        @pl.loop(0, rpw)
        def _(j):
            pltpu.sync_copy(msg_v.at[j], acc.at[idx_v.at[j]], add=True)
        plsc.subcore_barrier()
        pltpu.sync_copy(acc.at[pl.ds(sid * zr, zr)],
                        o_hbm.at[cid, pl.ds(sid * zr, zr)])

    return k(msg3, idx2)


def kernel(x, edge_index, edge_attr, batch, lin_W, lin_b, edge_W, edge_b,
           root_W, conv_b, gru_Wih, gru_Whh, gru_bih, gru_bhh,
           lstm_Wih, lstm_Whh, lstm_bih, lstm_bhh, sp_W, sp_b, prelu_a):
    f32 = jnp.float32
    n, node_in = x.shape
    e = edge_index.shape[1]
    h = lin_W.shape[1]
    k_in = edge_attr.shape[1]
    g = NUM_GRAPHS
    out_dim = sp_W.shape[1]
    ic = 2 * h

    chunk = NW * GW
    epad = ((e + chunk - 1) // chunk) * chunk
    pad = epad - e
    np_rows = ((n + 1 + 15) // 16) * 16
    src = edge_index[0]
    dst = edge_index[1]
    src2 = jnp.concatenate([src, jnp.zeros((pad,), jnp.int32)]).reshape(-1, GW)
    dst2 = jnp.concatenate([dst, jnp.full((pad,), n, jnp.int32)]).reshape(-1, GW)
    ea1 = jnp.concatenate([edge_attr, jnp.ones((e, 1), f32)], axis=1)
    ea_p = jnp.concatenate([ea1, jnp.zeros((pad, k_in + 1), f32)], axis=0)
    w2 = jnp.concatenate([edge_W, edge_b.reshape(1, h * h)],
                         axis=0).reshape((k_in + 1) * h, h)
    rm = jnp.kron(jnp.eye(k_in + 1, dtype=f32), jnp.ones((1, h), f32))
    wihT = gru_Wih.T
    whhT = gru_Whh.T
    bih2 = gru_bih.reshape(1, 3 * h)
    bhh2 = gru_bhh.reshape(1, 3 * h)
    lwihT = lstm_Wih.T
    lwhhT = lstm_Whh.T
    lb = (lstm_bih + lstm_bhh).reshape(1, 4 * ic)
    pa2 = jnp.broadcast_to(prelu_a.reshape(1, 1), (1, out_dim))

    node = pl.pallas_call(
        _node_body,
        out_shape=jax.ShapeDtypeStruct((n, h), f32),
    )(x, lin_W, lin_b.reshape(1, h))

    blk = 4096
    msg_call = pl.pallas_call(
        _msg_body,
        grid=(epad // blk,),
        in_specs=[
            pl.BlockSpec((blk, k_in + 1), lambda i: (i, 0)),
            pl.BlockSpec((blk, h), lambda i: (i, 0)),
            pl.BlockSpec((k_in + 1, (k_in + 1) * h), lambda i: (0, 0)),
            pl.BlockSpec(((k_in + 1) * h, h), lambda i: (0, 0)),
        ],
        out_specs=pl.BlockSpec((blk, h), lambda i: (i, 0)),
        out_shape=jax.ShapeDtypeStruct((epad, h), f32),
    )
    gru_call = pl.pallas_call(
        _gru_body,
        out_shape=jax.ShapeDtypeStruct((n, h), f32),
    )

    feats = node
    for _ in range(3):
        fs3 = _sc_gather(feats, src2)
        msg = msg_call(ea_p, fs3.reshape(epad, h), rm, w2)
        aggp = _sc_scatter_add(msg.reshape(-1, GW, h), dst2, np_rows)
        feats = gru_call(aggp, feats, feats, root_W, conv_b.reshape(1, h),
                         wihT, whhT, bih2, bhh2)

    na = jnp.concatenate([node, feats], axis=1)
    out = pl.pallas_call(
        functools.partial(_s2s_body, num_steps=3),
        out_shape=jax.ShapeDtypeStruct((g, out_dim), f32),
    )(na, na.T, batch.reshape(1, n), lwihT, lwhhT, lb,
      sp_W, sp_b.reshape(1, out_dim), pa2)
    return out


# speedup vs baseline: 3.9038x; 1.1684x over previous
import functools

import jax
import jax.numpy as jnp
from jax import lax
from jax.experimental import pallas as pl
from jax.experimental.pallas import tpu as pltpu
from jax.experimental.pallas import tpu_sc as plsc

NUM_GRAPHS = 256
GW = 128
NW = 32


def _node_body(x_ref, w_ref, b_ref, o_ref):
    o_ref[...] = jnp.maximum(x_ref[...] @ w_ref[...] + b_ref[...], 0.0)


def _msg_body(ea_ref, fs_ref, rm_ref, tt_ref, w2_ref, o_ref):
    ea = ea_ref[...]
    fs = fs_ref[...].astype(jnp.bfloat16)
    a = jnp.dot(ea, rm_ref[...],
                preferred_element_type=jnp.float32)
    b = jnp.dot(fs, tt_ref[...],
                preferred_element_type=jnp.float32)
    o_ref[...] = jnp.dot((a * b).astype(jnp.bfloat16), w2_ref[...],
                         preferred_element_type=jnp.float32)


def _gru_body(aggp_ref, f_ref, h_ref, rw_ref, cb_ref, wih_ref, whh_ref,
              bih_ref, bhh_ref, o_ref):
    n = f_ref.shape[0]
    hdim = f_ref.shape[1]
    agg = aggp_ref[0, :n, :] + aggp_ref[1, :n, :]
    f = f_ref[...]
    h = h_ref[...]
    conv = agg + f @ rw_ref[...] + cb_ref[...]
    a = jnp.maximum(conv, 0.0)
    gi = a @ wih_ref[...] + bih_ref[...]
    gh = h @ whh_ref[...] + bhh_ref[...]
    r = jax.nn.sigmoid(gi[:, :hdim] + gh[:, :hdim])
    z = jax.nn.sigmoid(gi[:, hdim:2 * hdim] + gh[:, hdim:2 * hdim])
    nn = jnp.tanh(gi[:, 2 * hdim:] + r * gh[:, 2 * hdim:])
    o_ref[...] = (1.0 - z) * nn + z * h


def _s2s_body(na_ref, naT_ref, b_ref, wih_ref, whh_ref, lb_ref,
              spw_ref, spb_ref, pa_ref, o_ref, num_steps):
    na = na_ref[...]
    naT = naT_ref[...]
    n = na.shape[0]
    ic = na.shape[1]
    g = NUM_GRAPHS
    brow = b_ref[...]
    gid = lax.broadcasted_iota(jnp.int32, (g, 1), 0)
    ot = brow == gid
    q_star = jnp.zeros((g, 2 * ic), jnp.float32)
    h_l = jnp.zeros((g, ic), jnp.float32)
    c_l = jnp.zeros((g, ic), jnp.float32)
    for _ in range(num_steps):
        gates = q_star @ wih_ref[...] + h_l @ whh_ref[...] + lb_ref[...]
        g_i = gates[:, :ic]
        g_f = gates[:, ic:2 * ic]
        g_g = gates[:, 2 * ic:3 * ic]
        g_o = gates[:, 3 * ic:]
        c_l = jax.nn.sigmoid(g_f) * c_l + jax.nn.sigmoid(g_i) * jnp.tanh(g_g)
        h_l = jax.nn.sigmoid(g_o) * jnp.tanh(c_l)
        d = h_l @ naT
        em = jnp.max(jnp.where(ot, d, -1e30), axis=1, keepdims=True)
        em = jnp.where(em > -1e29, em, 0.0)
        e2 = jnp.exp(jnp.where(ot, d - em, -1e30))
        den = jnp.sum(e2, axis=1, keepdims=True)
        alpha = e2 / (den + 1e-16)
        r_vec = alpha @ na
        q_star = jnp.concatenate([h_l, r_vec], axis=1)
    out = q_star @ spw_ref[...] + spb_ref[...]
    o_ref[...] = jnp.where(out > 0, out, pa_ref[...] * out)


_GRP = 8


def _sc_gather(table, idx2):
    nt, h = table.shape
    r = idx2.shape[0]
    rpw = r // NW
    mesh = plsc.VectorSubcoreMesh(core_axis_name="core",
                                  subcore_axis_name="subcore")

    @functools.partial(
        pl.kernel,
        out_type=jax.ShapeDtypeStruct((r * GW, h), jnp.float32),
        mesh=mesh,
        scratch_types=[pltpu.VMEM((rpw, GW), jnp.int32),
                       pltpu.VMEM((rpw * GW, h), jnp.float32),
                       pltpu.SemaphoreType.DMA],
        compiler_params=pltpu.CompilerParams(use_tc_tiling_on_sc=False),
    )
    def k(tab_hbm, idx_hbm, o_hbm, idx_v, rows_v, sem):
        cid = lax.axis_index("core")
        sid = lax.axis_index("subcore")
        wid = sid * 2 + cid
        r0 = wid * rpw
        pltpu.sync_copy(idx_hbm.at[pl.ds(r0, rpw)], idx_v)

        @pl.loop(0, rpw // _GRP)
        def _(t):
            cps = [pltpu.async_copy(
                tab_hbm.at[idx_v.at[t * _GRP + u]],
                rows_v.at[pl.ds((t * _GRP + u) * GW, GW)], sem)
                for u in range(_GRP)]
            for c in cps:
                c.wait()

        pltpu.sync_copy(rows_v, o_hbm.at[pl.ds(r0 * GW, rpw * GW)])

    return k(table, idx2)


def _sc_scatter_add(msg3, idx2, np_rows):
    r, _, h = msg3.shape
    rpw = r // NW
    zr = np_rows // 16
    mesh = plsc.VectorSubcoreMesh(core_axis_name="core",
                                  subcore_axis_name="subcore")

    @functools.partial(
        pl.kernel,
        out_type=jax.ShapeDtypeStruct((2, np_rows, h), jnp.float32),
        mesh=mesh,
        scratch_types=[pltpu.VMEM((rpw, GW), jnp.int32),
                       pltpu.VMEM((rpw, GW, h), jnp.float32),
                       pltpu.VMEM((zr, h), jnp.float32),
                       pltpu.VMEM_SHARED((np_rows, h), jnp.float32)],
        compiler_params=pltpu.CompilerParams(use_tc_tiling_on_sc=False),
    )
    def k(msg_hbm, idx_hbm, o_hbm, idx_v, msg_v, zb, acc):
        cid = lax.axis_index("core")
        sid = lax.axis_index("subcore")
        wid = sid * 2 + cid

        @pl.loop(0, zr)
        def _(i):
            zb[i] = jnp.zeros((h,), jnp.float32)

        pltpu.sync_copy(zb, acc.at[pl.ds(sid * zr, zr)])
        plsc.subcore_barrier()
        r0 = wid * rpw
        pltpu.sync_copy(idx_hbm.at[pl.ds(r0, rpw)], idx_v)
        pltpu.sync_copy(msg_hbm.at[pl.ds(r0, rpw)], msg_v)

        @pl.loop(0, rpw)
        def _(j):
            pltpu.sync_copy(msg_v.at[j], acc.at[idx_v.at[j]], add=True)
        plsc.subcore_barrier()
        pltpu.sync_copy(acc.at[pl.ds(sid * zr, zr)],
                        o_hbm.at[cid, pl.ds(sid * zr, zr)])

    return k(msg3, idx2)


def kernel(x, edge_index, edge_attr, batch, lin_W, lin_b, edge_W, edge_b,
           root_W, conv_b, gru_Wih, gru_Whh, gru_bih, gru_bhh,
           lstm_Wih, lstm_Whh, lstm_bih, lstm_bhh, sp_W, sp_b, prelu_a):
    f32 = jnp.float32
    n, node_in = x.shape
    e = edge_index.shape[1]
    h = lin_W.shape[1]
    k_in = edge_attr.shape[1]
    g = NUM_GRAPHS
    out_dim = sp_W.shape[1]
    ic = 2 * h

    chunk = NW * GW
    epad = ((e + chunk - 1) // chunk) * chunk
    pad = epad - e
    np_rows = ((n + 1 + 15) // 16) * 16
    src = edge_index[0]
    dst = edge_index[1]
    src2 = jnp.concatenate([src, jnp.zeros((pad,), jnp.int32)]).reshape(-1, GW)
    dst2 = jnp.concatenate([dst, jnp.full((pad,), n, jnp.int32)]).reshape(-1, GW)
    ea1 = jnp.concatenate([edge_attr, jnp.ones((e, 1), f32)], axis=1)
    ea_p = jnp.concatenate([ea1, jnp.zeros((pad, k_in + 1), f32)], axis=0)
    w2 = jnp.concatenate([edge_W, edge_b.reshape(1, h * h)],
                         axis=0).reshape((k_in + 1) * h, h)
    bf16 = jnp.bfloat16
    rm = jnp.kron(jnp.eye(k_in + 1, dtype=bf16), jnp.ones((1, h), bf16))
    tt = jnp.tile(jnp.eye(h, dtype=bf16), (1, k_in + 1))
    ea_pb = ea_p.astype(bf16)
    w2b = w2.astype(bf16)
    wihT = gru_Wih.T
    whhT = gru_Whh.T
    bih2 = gru_bih.reshape(1, 3 * h)
    bhh2 = gru_bhh.reshape(1, 3 * h)
    lwihT = lstm_Wih.T
    lwhhT = lstm_Whh.T
    lb = (lstm_bih + lstm_bhh).reshape(1, 4 * ic)
    pa2 = jnp.broadcast_to(prelu_a.reshape(1, 1), (1, out_dim))

    node = pl.pallas_call(
        _node_body,
        out_shape=jax.ShapeDtypeStruct((n, h), f32),
    )(x, lin_W, lin_b.reshape(1, h))

    blk = 4096
    msg_call = pl.pallas_call(
        _msg_body,
        grid=(epad // blk,),
        in_specs=[
            pl.BlockSpec((blk, k_in + 1), lambda i: (i, 0)),
            pl.BlockSpec((blk, h), lambda i: (i, 0)),
            pl.BlockSpec((k_in + 1, (k_in + 1) * h), lambda i: (0, 0)),
            pl.BlockSpec((h, (k_in + 1) * h), lambda i: (0, 0)),
            pl.BlockSpec(((k_in + 1) * h, h), lambda i: (0, 0)),
        ],
        out_specs=pl.BlockSpec((blk, h), lambda i: (i, 0)),
        out_shape=jax.ShapeDtypeStruct((epad, h), f32),
    )
    gru_call = pl.pallas_call(
        _gru_body,
        out_shape=jax.ShapeDtypeStruct((n, h), f32),
    )

    feats = node
    for _ in range(3):
        fs3 = _sc_gather(feats, src2)
        msg = msg_call(ea_pb, fs3.reshape(epad, h), rm, tt, w2b)
        aggp = _sc_scatter_add(msg.reshape(-1, GW, h), dst2, np_rows)
        feats = gru_call(aggp, feats, feats, root_W, conv_b.reshape(1, h),
                         wihT, whhT, bih2, bhh2)

    na = jnp.concatenate([node, feats], axis=1)
    out = pl.pallas_call(
        functools.partial(_s2s_body, num_steps=3),
        out_shape=jax.ShapeDtypeStruct((g, out_dim), f32),
    )(na, na.T, batch.reshape(1, n), lwihT, lwhhT, lb,
      sp_W, sp_b.reshape(1, out_dim), pa2)
    return out
